# baseline (speedup 1.0000x reference)
import functools

import jax
import jax.numpy as jnp
from jax.experimental import pallas as pl
from jax.experimental.pallas import tpu as pltpu

B = 128
L = 8
C = 3
R = L * C
IN_F = 900
D_MODEL = 768
E = 8
TOP_K = 2
LOW = 64
EPS = 1e-09

P = 8
NPAIR = B * TOP_K
NS = NPAIR + E * (P - 1)
NB = NS // P


def _routing_kernel(logits_ref, masks_ref, logits_t_ref, masks_t_ref,
                    selemb_ref, bias_ref,
                    se_ref, gb_ref, sid_ref, gate_ref, bstart_ref, bcount_ref):
    f32 = jnp.float32
    logits = logits_ref[...]
    mask = (masks_ref[...] == 1).astype(f32)
    p = jax.nn.softmax(logits, axis=1)
    g0 = p * mask
    inactive = p * (1.0 - mask)
    inorm = inactive / (jnp.sum(inactive, axis=1, keepdims=True) + EPS)
    se = jnp.zeros((B, LOW), f32)
    for e in range(E):
        se = se + selemb_ref[:, e, :] * inorm[:, e:e + 1]
    se_ref[...] = se

    iota_e = jax.lax.broadcasted_iota(jnp.int32, (B, E), 1)
    m1 = jnp.max(g0, axis=1, keepdims=True)
    i1 = jnp.min(jnp.where(g0 == m1, iota_e, E), axis=1, keepdims=True)
    g1 = jnp.where(iota_e == i1, -1.0, g0)
    m2 = jnp.max(g1, axis=1, keepdims=True)
    i2 = jnp.min(jnp.where(g1 == m2, iota_e, E), axis=1, keepdims=True)
    de = m1 + m2 + EPS
    w1 = m1 / de
    w2 = m2 / de
    gcomb = jnp.where(iota_e == i1, w1, 0.0) + jnp.where(iota_e == i2, w2, 0.0)
    gb_ref[...] = jnp.dot(gcomb, bias_ref[...],
                          preferred_element_type=f32)

    logits_t = logits_t_ref[...]
    mask_t = (masks_t_ref[...] == 1).astype(f32)
    p_t = jax.nn.softmax(logits_t, axis=0)
    g0t = p_t * mask_t
    iota_et = jax.lax.broadcasted_iota(jnp.int32, (E, B), 0)
    m1t = jnp.max(g0t, axis=0, keepdims=True)
    i1t = jnp.min(jnp.where(g0t == m1t, iota_et, E), axis=0, keepdims=True)
    g1t = jnp.where(iota_et == i1t, -1.0, g0t)
    m2t = jnp.max(g1t, axis=0, keepdims=True)
    i2t = jnp.min(jnp.where(g1t == m2t, iota_et, E), axis=0, keepdims=True)
    det = m1t + m2t + EPS
    w1t = m1t / det
    w2t = m2t / det

    e_row = jnp.concatenate([i1t, i2t], axis=1)
    w_row = jnp.concatenate([w1t, w2t], axis=1)
    b_row = jax.lax.broadcasted_iota(jnp.int32, (1, NPAIR), 1) % B

    onehot = (jnp.broadcast_to(e_row, (E, NPAIR)) ==
              jax.lax.broadcasted_iota(jnp.int32, (E, NPAIR), 0)).astype(f32)
    qi = jax.lax.broadcasted_iota(jnp.int32, (NPAIR, NPAIR), 0)
    qj = jax.lax.broadcasted_iota(jnp.int32, (NPAIR, NPAIR), 1)
    upper = (qi < qj).astype(f32)
    prefix = jnp.dot(onehot, upper, preferred_element_type=f32)
    rank = jnp.sum(onehot * prefix, axis=0, keepdims=True)

    cnt_col = jnp.sum(onehot, axis=1, keepdims=True)
    cntpad_col = jnp.floor((cnt_col + (P - 1)) * (1.0 / P)) * P
    eye = (jax.lax.broadcasted_iota(jnp.int32, (E, E), 0) ==
           jax.lax.broadcasted_iota(jnp.int32, (E, E), 1)).astype(f32)
    cntpad_row = jnp.dot(jnp.ones((1, E), f32), cntpad_col * eye,
                         preferred_element_type=f32)
    u8 = (jax.lax.broadcasted_iota(jnp.int32, (E, E), 0) <
          jax.lax.broadcasted_iota(jnp.int32, (E, E), 1)).astype(f32)
    offpad_row = jnp.dot(cntpad_row, u8, preferred_element_type=f32)
    endpad_row = offpad_row + cntpad_row

    l8 = (jax.lax.broadcasted_iota(jnp.int32, (E, E), 0) >
          jax.lax.broadcasted_iota(jnp.int32, (E, E), 1)).astype(f32)
    offpad_col = jnp.dot(l8, cntpad_col, preferred_element_type=f32)
    off_sel = jnp.sum(onehot * jnp.broadcast_to(offpad_col, (E, NPAIR)),
                      axis=0, keepdims=True)
    pos = off_sel + rank

    slot = jax.lax.broadcasted_iota(jnp.int32, (NS, NPAIR), 0).astype(f32)
    hit = (slot == jnp.broadcast_to(pos, (NS, NPAIR))).astype(f32)
    sid_ref[...] = jnp.sum(hit * b_row.astype(f32), axis=1,
                           keepdims=True).astype(jnp.int32)
    gate_ref[...] = jnp.sum(hit * w_row, axis=1, keepdims=True)

    bstart_ref[...] = (offpad_col * (1.0 / P)).astype(jnp.int32)
    bcount_ref[...] = (cntpad_col * (1.0 / P)).astype(jnp.int32)


def _moe_kernel(sid_ref, gate_ref, bstart_ref, bcount_ref,
                x_ref, w_ref, gb_ref, out_ref,
                acc_ref, wt_ref, xg_ref, y_ref):
    acc_ref[...] = jnp.broadcast_to(gb_ref[...][:, None, :], (B, R, D_MODEL))

    for e in range(E):
        wt_ref[...] = jnp.transpose(w_ref[e], (1, 0))
        base = bstart_ref[e, 0]

        def block_body(j, carry):
            jj = base + j
            for p_i in range(P):
                s = sid_ref[jj * P + p_i, 0]
                xg_ref[p_i * R:(p_i + 1) * R, :] = x_ref[s]
            y_ref[...] = jax.lax.dot_general(
                xg_ref[...], wt_ref[...], (((1,), (0,)), ((), ())),
                precision=jax.lax.Precision.DEFAULT,
                preferred_element_type=jnp.float32)
            for p_i in range(P):
                s = sid_ref[jj * P + p_i, 0]
                gval = gate_ref[jj * P + p_i, 0]
                acc_ref[s] = acc_ref[s] + gval * y_ref[p_i * R:(p_i + 1) * R, :]
            return carry

        jax.lax.fori_loop(0, bcount_ref[e, 0], block_body, 0)

    out_ref[...] = acc_ref[...].astype(jnp.bfloat16)


@functools.partial(jax.jit, static_argnames=())
def kernel(cycle_curve_data, logits, moe_masks, selection_embeddings, W, b):
    f32 = jnp.float32
    x3 = cycle_curve_data.reshape(B, R, IN_F)

    se, gb, sid, gate, bstart, bcount = pl.pallas_call(
        _routing_kernel,
        out_shape=(
            jax.ShapeDtypeStruct((B, LOW), f32),
            jax.ShapeDtypeStruct((B, D_MODEL), f32),
            jax.ShapeDtypeStruct((NS, 1), jnp.int32),
            jax.ShapeDtypeStruct((NS, 1), f32),
            jax.ShapeDtypeStruct((E, 1), jnp.int32),
            jax.ShapeDtypeStruct((E, 1), jnp.int32),
        ),
    )(logits, moe_masks, jnp.transpose(logits), jnp.transpose(moe_masks),
      selection_embeddings, b)

    out = pl.pallas_call(
        _moe_kernel,
        in_specs=[
            pl.BlockSpec(memory_space=pltpu.SMEM),
            pl.BlockSpec(memory_space=pltpu.SMEM),
            pl.BlockSpec(memory_space=pltpu.SMEM),
            pl.BlockSpec(memory_space=pltpu.SMEM),
            pl.BlockSpec(memory_space=pltpu.VMEM),
            pl.BlockSpec(memory_space=pltpu.VMEM),
            pl.BlockSpec(memory_space=pltpu.VMEM),
        ],
        out_shape=jax.ShapeDtypeStruct((B, R, D_MODEL), jnp.bfloat16),
        scratch_shapes=[
            pltpu.VMEM((B, R, D_MODEL), f32),
            pltpu.VMEM((IN_F, D_MODEL), f32),
            pltpu.VMEM((P * R, IN_F), f32),
            pltpu.VMEM((P * R, D_MODEL), f32),
        ],
    )(sid, gate, bstart, bcount, x3, W, gb)

    final_out = out.reshape(B, L, C, D_MODEL)
    return (final_out, jnp.float32(0.0), se)


# speedup vs baseline: 1.0013x; 1.0013x over previous
import functools

import jax
import jax.numpy as jnp
from jax.experimental import pallas as pl
from jax.experimental.pallas import tpu as pltpu

B = 128
L = 8
C = 3
R = L * C
IN_F = 900
D_MODEL = 768
E = 8
TOP_K = 2
LOW = 64
EPS = 1e-09

P = 8
PR = P * R
NPAIR = B * TOP_K
NS = NPAIR + E * (P - 1)
NB = NS // P


def _routing_kernel(logits_ref, masks_ref, selemb_ref, bias_ref,
                    se_ref, gb_ref, sid_ref, ebid_ref, nblk_ref,
                    pos_ref, gw_ref):
    f32 = jnp.float32
    logits = logits_ref[...]
    mask = (masks_ref[...] == 1).astype(f32)
    p = jax.nn.softmax(logits, axis=1)
    g0 = p * mask
    inactive = p * (1.0 - mask)
    inorm = inactive / (jnp.sum(inactive, axis=1, keepdims=True) + EPS)
    se = jnp.zeros((B, LOW), f32)
    for e in range(E):
        se = se + selemb_ref[:, e, :] * inorm[:, e:e + 1]
    se_ref[...] = se

    iota_e = jax.lax.broadcasted_iota(jnp.int32, (B, E), 1)
    m1 = jnp.max(g0, axis=1, keepdims=True)
    i1 = jnp.min(jnp.where(g0 == m1, iota_e, E), axis=1, keepdims=True)
    g1 = jnp.where(iota_e == i1, -1.0, g0)
    m2 = jnp.max(g1, axis=1, keepdims=True)
    i2 = jnp.min(jnp.where(g1 == m2, iota_e, E), axis=1, keepdims=True)
    de = m1 + m2 + EPS
    w1 = m1 / de
    w2 = m2 / de
    gcomb = jnp.where(iota_e == i1, w1, 0.0) + jnp.where(iota_e == i2, w2, 0.0)
    gb_ref[...] = jnp.dot(gcomb, bias_ref[...], preferred_element_type=f32)

    logits_t = jnp.transpose(logits, (1, 0))
    mask_t = jnp.transpose(mask, (1, 0))
    p_t = jax.nn.softmax(logits_t, axis=0)
    g0t = p_t * mask_t
    iota_et = jax.lax.broadcasted_iota(jnp.int32, (E, B), 0)
    m1t = jnp.max(g0t, axis=0, keepdims=True)
    i1t = jnp.min(jnp.where(g0t == m1t, iota_et, E), axis=0, keepdims=True)
    g1t = jnp.where(iota_et == i1t, -1.0, g0t)
    m2t = jnp.max(g1t, axis=0, keepdims=True)
    i2t = jnp.min(jnp.where(g1t == m2t, iota_et, E), axis=0, keepdims=True)
    det = m1t + m2t + EPS
    w1t = m1t / det
    w2t = m2t / det

    e_row = jnp.concatenate([i1t, i2t], axis=1)
    w_row = jnp.concatenate([w1t, w2t], axis=1)
    b_row = jax.lax.broadcasted_iota(jnp.int32, (1, NPAIR), 1) % B
    gw_ref[...] = w_row

    onehot = (jnp.broadcast_to(e_row, (E, NPAIR)) ==
              jax.lax.broadcasted_iota(jnp.int32, (E, NPAIR), 0)).astype(f32)
    qi = jax.lax.broadcasted_iota(jnp.int32, (NPAIR, NPAIR), 0)
    qj = jax.lax.broadcasted_iota(jnp.int32, (NPAIR, NPAIR), 1)
    upper = (qi < qj).astype(f32)
    prefix = jnp.dot(onehot, upper, preferred_element_type=f32)
    rank = jnp.sum(onehot * prefix, axis=0, keepdims=True)

    cnt_col = jnp.sum(onehot, axis=1, keepdims=True)
    cntpad_col = jnp.floor((cnt_col + (P - 1)) * (1.0 / P)) * P
    eye = (jax.lax.broadcasted_iota(jnp.int32, (E, E), 0) ==
           jax.lax.broadcasted_iota(jnp.int32, (E, E), 1)).astype(f32)
    cntpad_row = jnp.dot(jnp.ones((1, E), f32), cntpad_col * eye,
                         preferred_element_type=f32)
    u8 = (jax.lax.broadcasted_iota(jnp.int32, (E, E), 0) <
          jax.lax.broadcasted_iota(jnp.int32, (E, E), 1)).astype(f32)
    offpad_row = jnp.dot(cntpad_row, u8, preferred_element_type=f32)
    endpad_row = offpad_row + cntpad_row

    l8 = (jax.lax.broadcasted_iota(jnp.int32, (E, E), 0) >
          jax.lax.broadcasted_iota(jnp.int32, (E, E), 1)).astype(f32)
    offpad_col = jnp.dot(l8, cntpad_col, preferred_element_type=f32)
    off_sel = jnp.sum(onehot * jnp.broadcast_to(offpad_col, (E, NPAIR)),
                      axis=0, keepdims=True)
    pos = off_sel + rank
    pos_ref[...] = pos.astype(jnp.int32)

    slot = jax.lax.broadcasted_iota(jnp.int32, (NS, NPAIR), 0).astype(f32)
    hit = (slot == jnp.broadcast_to(pos, (NS, NPAIR))).astype(f32)
    sid_ref[...] = jnp.sum(hit * b_row.astype(f32), axis=1,
                           keepdims=True).astype(jnp.int32)

    blk = (jax.lax.broadcasted_iota(jnp.int32, (NB, E), 0) * P).astype(f32)
    endb = jnp.broadcast_to(endpad_row, (NB, E))
    ebid = jnp.sum((blk >= endb).astype(f32), axis=1, keepdims=True)
    ebid_ref[...] = jnp.minimum(ebid, E - 1).astype(jnp.int32)
    nblk_ref[...] = (jnp.sum(cntpad_col * (1.0 / P))
                     .reshape(1, 1).astype(jnp.int32))


def _moe_kernel(sid_ref, ebid_ref, nblk_ref, pos_ref, gw_ref,
                x_ref, w_ref, gb_ref, out_ref,
                wt_ref, xg_ref, ys_ref, sem_ref):
    f32 = jnp.float32
    nb = nblk_ref[0, 0]

    def start_gather(j, buf):
        for p_i in range(P):
            s = sid_ref[j * P + p_i, 0]
            pltpu.make_async_copy(
                x_ref.at[s], xg_ref.at[buf, pl.ds(p_i * R, R), :],
                sem_ref.at[buf]).start()

    def wait_gather(j, buf):
        for p_i in range(P):
            s = sid_ref[j * P + p_i, 0]
            pltpu.make_async_copy(
                x_ref.at[s], xg_ref.at[buf, pl.ds(p_i * R, R), :],
                sem_ref.at[buf]).wait()

    start_gather(0, 0)

    for e in range(E):
        wt_ref[e] = jnp.transpose(w_ref[e], (1, 0))

    def block_body(j, carry):
        buf = jax.lax.rem(j, 2)
        wait_gather(j, buf)

        @pl.when(j + 1 < nb)
        def _prefetch():
            start_gather(j + 1, 1 - buf)

        e = ebid_ref[j, 0]
        y = jax.lax.dot_general(
            xg_ref[buf], wt_ref[e], (((1,), (0,)), ((), ())),
            preferred_element_type=f32)
        ys_ref[pl.ds(j * PR, PR), :] = y.astype(jnp.bfloat16)
        return carry

    jax.lax.fori_loop(0, nb, block_body, 0)

    def combine_body(bi, carry):
        p0 = pos_ref[0, bi]
        p1 = pos_ref[0, B + bi]
        g0 = gw_ref[0, bi]
        g1 = gw_ref[0, B + bi]
        y0 = ys_ref[pl.ds(p0 * R, R), :].astype(f32)
        y1 = ys_ref[pl.ds(p1 * R, R), :].astype(f32)
        out_ref[bi] = (gb_ref[bi][None, :] + g0 * y0
                       + g1 * y1).astype(jnp.bfloat16)
        return carry

    jax.lax.fori_loop(0, B, combine_body, 0)


@functools.partial(jax.jit, static_argnames=())
def kernel(cycle_curve_data, logits, moe_masks, selection_embeddings, W, b):
    f32 = jnp.float32
    x3 = cycle_curve_data.reshape(B, R, IN_F).astype(jnp.bfloat16)
    wb = W.astype(jnp.bfloat16)

    se, gb, sid, ebid, nblk, pos, gw = pl.pallas_call(
        _routing_kernel,
        out_shape=(
            jax.ShapeDtypeStruct((B, LOW), f32),
            jax.ShapeDtypeStruct((B, D_MODEL), f32),
            jax.ShapeDtypeStruct((NS, 1), jnp.int32),
            jax.ShapeDtypeStruct((NB, 1), jnp.int32),
            jax.ShapeDtypeStruct((1, 1), jnp.int32),
            jax.ShapeDtypeStruct((1, NPAIR), jnp.int32),
            jax.ShapeDtypeStruct((1, NPAIR), f32),
        ),
    )(logits, moe_masks, selection_embeddings, b)

    out = pl.pallas_call(
        _moe_kernel,
        in_specs=[
            pl.BlockSpec(memory_space=pltpu.SMEM),
            pl.BlockSpec(memory_space=pltpu.SMEM),
            pl.BlockSpec(memory_space=pltpu.SMEM),
            pl.BlockSpec(memory_space=pltpu.SMEM),
            pl.BlockSpec(memory_space=pltpu.SMEM),
            pl.BlockSpec(memory_space=pltpu.VMEM),
            pl.BlockSpec(memory_space=pltpu.VMEM),
            pl.BlockSpec(memory_space=pltpu.VMEM),
        ],
        out_shape=jax.ShapeDtypeStruct((B, R, D_MODEL), jnp.bfloat16),
        scratch_shapes=[
            pltpu.VMEM((E, IN_F, D_MODEL), jnp.bfloat16),
            pltpu.VMEM((2, PR, IN_F), jnp.bfloat16),
            pltpu.VMEM((NS * R, D_MODEL), jnp.bfloat16),
            pltpu.SemaphoreType.DMA((2,)),
        ],
    )(sid, ebid, nblk, pos, gw, x3, wb, gb)

    final_out = out.reshape(B, L, C, D_MODEL)
    return (final_out, jnp.float32(0.0), se)


# speedup vs baseline: 1.0849x; 1.0835x over previous
import functools

import jax
import jax.numpy as jnp
from jax.experimental import pallas as pl
from jax.experimental.pallas import tpu as pltpu

B = 128
L = 8
C = 3
R = L * C
IN_F = 900
D_MODEL = 768
E = 8
TOP_K = 2
LOW = 64
EPS = 1e-09

P = 8
PR = P * R
NPAIR = B * TOP_K
NS = NPAIR + E * (P - 1)
NB = NS // P


def _routing_kernel(logits_ref, masks_ref, selemb_ref, bias_ref,
                    se_ref, gb_ref, sid_ref, ebid_ref, nblk_ref,
                    pos_ref, gw_ref):
    f32 = jnp.float32
    logits = logits_ref[...]
    mask = (masks_ref[...] == 1).astype(f32)
    p = jax.nn.softmax(logits, axis=1)
    g0 = p * mask
    inactive = p * (1.0 - mask)
    inorm = inactive / (jnp.sum(inactive, axis=1, keepdims=True) + EPS)
    se = jnp.zeros((B, LOW), f32)
    for e in range(E):
        se = se + selemb_ref[:, e, :] * inorm[:, e:e + 1]
    se_ref[...] = se

    iota_e = jax.lax.broadcasted_iota(jnp.int32, (B, E), 1)
    m1 = jnp.max(g0, axis=1, keepdims=True)
    i1 = jnp.min(jnp.where(g0 == m1, iota_e, E), axis=1, keepdims=True)
    g1 = jnp.where(iota_e == i1, -1.0, g0)
    m2 = jnp.max(g1, axis=1, keepdims=True)
    i2 = jnp.min(jnp.where(g1 == m2, iota_e, E), axis=1, keepdims=True)
    de = m1 + m2 + EPS
    w1 = m1 / de
    w2 = m2 / de
    gcomb = jnp.where(iota_e == i1, w1, 0.0) + jnp.where(iota_e == i2, w2, 0.0)
    gb_ref[...] = jnp.dot(gcomb, bias_ref[...], preferred_element_type=f32)

    logits_t = jnp.transpose(logits, (1, 0))
    mask_t = jnp.transpose(mask, (1, 0))
    p_t = jax.nn.softmax(logits_t, axis=0)
    g0t = p_t * mask_t
    iota_et = jax.lax.broadcasted_iota(jnp.int32, (E, B), 0)
    m1t = jnp.max(g0t, axis=0, keepdims=True)
    i1t = jnp.min(jnp.where(g0t == m1t, iota_et, E), axis=0, keepdims=True)
    g1t = jnp.where(iota_et == i1t, -1.0, g0t)
    m2t = jnp.max(g1t, axis=0, keepdims=True)
    i2t = jnp.min(jnp.where(g1t == m2t, iota_et, E), axis=0, keepdims=True)
    det = m1t + m2t + EPS
    w1t = m1t / det
    w2t = m2t / det

    e_row = jnp.concatenate([i1t, i2t], axis=1)
    w_row = jnp.concatenate([w1t, w2t], axis=1)
    b_row = jax.lax.broadcasted_iota(jnp.int32, (1, NPAIR), 1) % B
    gw_ref[...] = w_row

    onehot = (jnp.broadcast_to(e_row, (E, NPAIR)) ==
              jax.lax.broadcasted_iota(jnp.int32, (E, NPAIR), 0)).astype(f32)
    qi = jax.lax.broadcasted_iota(jnp.int32, (NPAIR, NPAIR), 0)
    qj = jax.lax.broadcasted_iota(jnp.int32, (NPAIR, NPAIR), 1)
    upper = (qi < qj).astype(f32)
    prefix = jnp.dot(onehot, upper, preferred_element_type=f32)
    rank = jnp.sum(onehot * prefix, axis=0, keepdims=True)

    cnt_col = jnp.sum(onehot, axis=1, keepdims=True)
    cntpad_col = jnp.floor((cnt_col + (P - 1)) * (1.0 / P)) * P
    eye = (jax.lax.broadcasted_iota(jnp.int32, (E, E), 0) ==
           jax.lax.broadcasted_iota(jnp.int32, (E, E), 1)).astype(f32)
    cntpad_row = jnp.dot(jnp.ones((1, E), f32), cntpad_col * eye,
                         preferred_element_type=f32)
    u8 = (jax.lax.broadcasted_iota(jnp.int32, (E, E), 0) <
          jax.lax.broadcasted_iota(jnp.int32, (E, E), 1)).astype(f32)
    offpad_row = jnp.dot(cntpad_row, u8, preferred_element_type=f32)
    endpad_row = offpad_row + cntpad_row

    l8 = (jax.lax.broadcasted_iota(jnp.int32, (E, E), 0) >
          jax.lax.broadcasted_iota(jnp.int32, (E, E), 1)).astype(f32)
    offpad_col = jnp.dot(l8, cntpad_col, preferred_element_type=f32)
    off_sel = jnp.sum(onehot * jnp.broadcast_to(offpad_col, (E, NPAIR)),
                      axis=0, keepdims=True)
    pos = off_sel + rank
    pos_ref[...] = pos.astype(jnp.int32)

    slot = jax.lax.broadcasted_iota(jnp.int32, (NS, NPAIR), 0).astype(f32)
    hit = (slot == jnp.broadcast_to(pos, (NS, NPAIR))).astype(f32)
    sid_ref[...] = jnp.sum(hit * b_row.astype(f32), axis=1,
                           keepdims=True).astype(jnp.int32)

    blk = (jax.lax.broadcasted_iota(jnp.int32, (NB, E), 0) * P).astype(f32)
    endb = jnp.broadcast_to(endpad_row, (NB, E))
    ebid = jnp.sum((blk >= endb).astype(f32), axis=1, keepdims=True)
    ebid_ref[...] = jnp.minimum(ebid, E - 1).astype(jnp.int32)
    nblk_ref[...] = (jnp.sum(cntpad_col * (1.0 / P))
                     .reshape(1, 1).astype(jnp.int32))


def _moe_kernel(sid_ref, ebid_ref, nblk_ref, pos_ref, gw_ref,
                x_ref, wt_ref, gb_ref, out_ref,
                xg_ref, ys_ref):
    f32 = jnp.float32
    nb = nblk_ref[0, 0]

    def gather_block(j):
        for p_i in range(P):
            s = sid_ref[j * P + p_i, 0]
            xg_ref[pl.ds(j * PR + p_i * R, R), :] = (
                x_ref[s].astype(jnp.bfloat16))

    gather_block(0)

    def block_body(j, carry):
        @pl.when(j + 1 < nb)
        def _prefetch():
            gather_block(j + 1)

        e = ebid_ref[j, 0]
        y = jax.lax.dot_general(
            xg_ref[pl.ds(j * PR, PR), :], wt_ref[e], (((1,), (0,)), ((), ())),
            preferred_element_type=f32)
        ys_ref[pl.ds(j * PR, PR), :] = y.astype(jnp.bfloat16)
        return carry

    jax.lax.fori_loop(0, nb, block_body, 0)

    def combine_body(bi, carry):
        p0 = pos_ref[0, bi]
        p1 = pos_ref[0, B + bi]
        g0 = gw_ref[0, bi]
        g1 = gw_ref[0, B + bi]
        y0 = ys_ref[pl.ds(p0 * R, R), :].astype(f32)
        y1 = ys_ref[pl.ds(p1 * R, R), :].astype(f32)
        out_ref[bi] = (gb_ref[bi][None, :] + g0 * y0
                       + g1 * y1).astype(jnp.bfloat16)
        return carry

    jax.lax.fori_loop(0, B, combine_body, 0)


@functools.partial(jax.jit, static_argnames=())
def kernel(cycle_curve_data, logits, moe_masks, selection_embeddings, W, b):
    f32 = jnp.float32
    x3 = cycle_curve_data.reshape(B, R, IN_F)
    wt = jnp.transpose(W, (0, 2, 1)).astype(jnp.bfloat16)

    se, gb, sid, ebid, nblk, pos, gw = pl.pallas_call(
        _routing_kernel,
        out_shape=(
            jax.ShapeDtypeStruct((B, LOW), f32),
            jax.ShapeDtypeStruct((B, D_MODEL), f32),
            jax.ShapeDtypeStruct((NS, 1), jnp.int32),
            jax.ShapeDtypeStruct((NB, 1), jnp.int32),
            jax.ShapeDtypeStruct((1, 1), jnp.int32),
            jax.ShapeDtypeStruct((1, NPAIR), jnp.int32),
            jax.ShapeDtypeStruct((1, NPAIR), f32),
        ),
    )(logits, moe_masks, selection_embeddings, b)

    out = pl.pallas_call(
        _moe_kernel,
        in_specs=[
            pl.BlockSpec(memory_space=pltpu.SMEM),
            pl.BlockSpec(memory_space=pltpu.SMEM),
            pl.BlockSpec(memory_space=pltpu.SMEM),
            pl.BlockSpec(memory_space=pltpu.SMEM),
            pl.BlockSpec(memory_space=pltpu.SMEM),
            pl.BlockSpec(memory_space=pltpu.VMEM),
            pl.BlockSpec(memory_space=pltpu.VMEM),
            pl.BlockSpec(memory_space=pltpu.VMEM),
        ],
        out_shape=jax.ShapeDtypeStruct((B, R, D_MODEL), jnp.bfloat16),
        scratch_shapes=[
            pltpu.VMEM((NS * R, IN_F), jnp.bfloat16),
            pltpu.VMEM((NS * R, D_MODEL), jnp.bfloat16),
        ],
    )(sid, ebid, nblk, pos, gw, x3, wt, gb)

    final_out = out.reshape(B, L, C, D_MODEL)
    return (final_out, jnp.float32(0.0), se)


# speedup vs baseline: 1.2147x; 1.1197x over previous
import functools

import jax
import jax.numpy as jnp
from jax.experimental import pallas as pl
from jax.experimental.pallas import tpu as pltpu

B = 128
L = 8
C = 3
R = L * C
IN_F = 900
D_MODEL = 768
E = 8
TOP_K = 2
LOW = 64
EPS = 1e-09

P = 8
PR = P * R
RPAD = 32
NPAIR = B * TOP_K
NS = NPAIR + E * (P - 1)
NB = NS // P


def _routing_kernel(logits_ref, masks_ref, selemb_ref, bias_ref,
                    se_ref, gb_ref, sid_ref, ebid_ref, nblk_ref,
                    pos_ref, gw_ref):
    f32 = jnp.float32
    logits = logits_ref[...]
    mask = (masks_ref[...] == 1).astype(f32)
    p = jax.nn.softmax(logits, axis=1)
    g0 = p * mask
    inactive = p * (1.0 - mask)
    inorm = inactive / (jnp.sum(inactive, axis=1, keepdims=True) + EPS)
    se = jnp.zeros((B, LOW), f32)
    for e in range(E):
        se = se + selemb_ref[:, e, :] * inorm[:, e:e + 1]
    se_ref[...] = se

    iota_e = jax.lax.broadcasted_iota(jnp.int32, (B, E), 1)
    m1 = jnp.max(g0, axis=1, keepdims=True)
    i1 = jnp.min(jnp.where(g0 == m1, iota_e, E), axis=1, keepdims=True)
    g1 = jnp.where(iota_e == i1, -1.0, g0)
    m2 = jnp.max(g1, axis=1, keepdims=True)
    i2 = jnp.min(jnp.where(g1 == m2, iota_e, E), axis=1, keepdims=True)
    de = m1 + m2 + EPS
    w1 = m1 / de
    w2 = m2 / de
    gcomb = jnp.where(iota_e == i1, w1, 0.0) + jnp.where(iota_e == i2, w2, 0.0)
    gb_ref[...] = jnp.dot(gcomb, bias_ref[...], preferred_element_type=f32)

    logits_t = jnp.transpose(logits, (1, 0))
    mask_t = jnp.transpose(mask, (1, 0))
    p_t = jax.nn.softmax(logits_t, axis=0)
    g0t = p_t * mask_t
    iota_et = jax.lax.broadcasted_iota(jnp.int32, (E, B), 0)
    m1t = jnp.max(g0t, axis=0, keepdims=True)
    i1t = jnp.min(jnp.where(g0t == m1t, iota_et, E), axis=0, keepdims=True)
    g1t = jnp.where(iota_et == i1t, -1.0, g0t)
    m2t = jnp.max(g1t, axis=0, keepdims=True)
    i2t = jnp.min(jnp.where(g1t == m2t, iota_et, E), axis=0, keepdims=True)
    det = m1t + m2t + EPS
    w1t = m1t / det
    w2t = m2t / det

    e_row = jnp.concatenate([i1t, i2t], axis=1)
    w_row = jnp.concatenate([w1t, w2t], axis=1)
    b_row = jax.lax.broadcasted_iota(jnp.int32, (1, NPAIR), 1) % B
    gw_ref[...] = w_row

    onehot = (jnp.broadcast_to(e_row, (E, NPAIR)) ==
              jax.lax.broadcasted_iota(jnp.int32, (E, NPAIR), 0)).astype(f32)
    qi = jax.lax.broadcasted_iota(jnp.int32, (NPAIR, NPAIR), 0)
    qj = jax.lax.broadcasted_iota(jnp.int32, (NPAIR, NPAIR), 1)
    upper = (qi < qj).astype(f32)
    prefix = jnp.dot(onehot, upper, preferred_element_type=f32)
    rank = jnp.sum(onehot * prefix, axis=0, keepdims=True)

    cnt_col = jnp.sum(onehot, axis=1, keepdims=True)
    cntpad_col = jnp.floor((cnt_col + (P - 1)) * (1.0 / P)) * P
    eye = (jax.lax.broadcasted_iota(jnp.int32, (E, E), 0) ==
           jax.lax.broadcasted_iota(jnp.int32, (E, E), 1)).astype(f32)
    cntpad_row = jnp.dot(jnp.ones((1, E), f32), cntpad_col * eye,
                         preferred_element_type=f32)
    u8 = (jax.lax.broadcasted_iota(jnp.int32, (E, E), 0) <
          jax.lax.broadcasted_iota(jnp.int32, (E, E), 1)).astype(f32)
    offpad_row = jnp.dot(cntpad_row, u8, preferred_element_type=f32)
    endpad_row = offpad_row + cntpad_row

    l8 = (jax.lax.broadcasted_iota(jnp.int32, (E, E), 0) >
          jax.lax.broadcasted_iota(jnp.int32, (E, E), 1)).astype(f32)
    offpad_col = jnp.dot(l8, cntpad_col, preferred_element_type=f32)
    off_sel = jnp.sum(onehot * jnp.broadcast_to(offpad_col, (E, NPAIR)),
                      axis=0, keepdims=True)
    pos = off_sel + rank
    pos_ref[...] = pos.astype(jnp.int32)

    slot = jax.lax.broadcasted_iota(jnp.int32, (NS, NPAIR), 0).astype(f32)
    hit = (slot == jnp.broadcast_to(pos, (NS, NPAIR))).astype(f32)
    sid_ref[...] = jnp.sum(hit * b_row.astype(f32), axis=1,
                           keepdims=True).astype(jnp.int32)

    blk = (jax.lax.broadcasted_iota(jnp.int32, (NB, E), 0) * P).astype(f32)
    endb = jnp.broadcast_to(endpad_row, (NB, E))
    ebid = jnp.sum((blk >= endb).astype(f32), axis=1, keepdims=True)
    ebid_ref[...] = jnp.minimum(ebid, E - 1).astype(jnp.int32)
    nblk_ref[...] = (jnp.sum(cntpad_col * (1.0 / P))
                     .reshape(1, 1).astype(jnp.int32))


def _moe_kernel(sid_ref, ebid_ref, nblk_ref, pos_ref, gw_ref,
                x_ref, wt_ref, gb_ref, out_ref,
                xg_ref, ys_ref):
    f32 = jnp.float32
    nb = nblk_ref[0, 0]

    def gather_block(j, buf):
        for p_i in range(P):
            s = sid_ref[j * P + p_i, 0]
            xg_ref[buf, pl.ds(p_i * R, R), :] = x_ref[s]

    gather_block(0, 0)

    def block_body(j, carry):
        buf = jax.lax.rem(j, 2)

        @pl.when(j + 1 < nb)
        def _prefetch():
            gather_block(j + 1, 1 - buf)

        e = ebid_ref[j, 0]
        y = jax.lax.dot_general(
            xg_ref[buf], wt_ref[e], (((1,), (0,)), ((), ())),
            preferred_element_type=f32)
        for p_i in range(P):
            ys_ref[pl.ds((j * P + p_i) * RPAD, R), :] = (
                y[p_i * R:(p_i + 1) * R, :].astype(jnp.bfloat16))
        return carry

    jax.lax.fori_loop(0, nb, block_body, 0)

    def combine_body(bi, carry):
        p0 = pos_ref[0, bi]
        p1 = pos_ref[0, B + bi]
        g0 = gw_ref[0, bi]
        g1 = gw_ref[0, B + bi]
        y0 = ys_ref[pl.ds(p0 * RPAD, R), :].astype(f32)
        y1 = ys_ref[pl.ds(p1 * RPAD, R), :].astype(f32)
        out_ref[bi] = (gb_ref[bi][None, :] + g0 * y0
                       + g1 * y1).astype(jnp.bfloat16)
        return carry

    jax.lax.fori_loop(0, B, combine_body, 0)


@functools.partial(jax.jit, static_argnames=())
def kernel(cycle_curve_data, logits, moe_masks, selection_embeddings, W, b):
    f32 = jnp.float32
    x3 = cycle_curve_data.reshape(B, R, IN_F).astype(jnp.bfloat16)
    wt = jnp.transpose(W, (0, 2, 1)).astype(jnp.bfloat16)

    se, gb, sid, ebid, nblk, pos, gw = pl.pallas_call(
        _routing_kernel,
        out_shape=(
            jax.ShapeDtypeStruct((B, LOW), f32),
            jax.ShapeDtypeStruct((B, D_MODEL), f32),
            jax.ShapeDtypeStruct((NS, 1), jnp.int32),
            jax.ShapeDtypeStruct((NB, 1), jnp.int32),
            jax.ShapeDtypeStruct((1, 1), jnp.int32),
            jax.ShapeDtypeStruct((1, NPAIR), jnp.int32),
            jax.ShapeDtypeStruct((1, NPAIR), f32),
        ),
    )(logits, moe_masks, selection_embeddings, b)

    out = pl.pallas_call(
        _moe_kernel,
        in_specs=[
            pl.BlockSpec(memory_space=pltpu.SMEM),
            pl.BlockSpec(memory_space=pltpu.SMEM),
            pl.BlockSpec(memory_space=pltpu.SMEM),
            pl.BlockSpec(memory_space=pltpu.SMEM),
            pl.BlockSpec(memory_space=pltpu.SMEM),
            pl.BlockSpec(memory_space=pltpu.VMEM),
            pl.BlockSpec(memory_space=pltpu.VMEM),
            pl.BlockSpec(memory_space=pltpu.VMEM),
        ],
        out_shape=jax.ShapeDtypeStruct((B, R, D_MODEL), jnp.bfloat16),
        scratch_shapes=[
            pltpu.VMEM((2, PR, IN_F), jnp.bfloat16),
            pltpu.VMEM((NS * RPAD, D_MODEL), jnp.bfloat16),
        ],
    )(sid, ebid, nblk, pos, gw, x3, wt, gb)

    final_out = out.reshape(B, L, C, D_MODEL)
    return (final_out, jnp.float32(0.0), se)


# speedup vs baseline: 1.2196x; 1.0040x over previous
import functools

import jax
import jax.numpy as jnp
from jax.experimental import pallas as pl
from jax.experimental.pallas import tpu as pltpu

B = 128
L = 8
C = 3
R = L * C
IN_F = 900
D_MODEL = 768
E = 8
TOP_K = 2
LOW = 64
EPS = 1e-09

P = 8
PR = P * R
RPAD = 32
NPAIR = B * TOP_K
NS = NPAIR + E * (P - 1)
NB = NS // P


def _routing_kernel(logits_ref, masks_ref, selemb_ref, bias_ref,
                    se_ref, gb_ref, sid_ref, ebid_ref, nblk_ref,
                    pos_ref, gw_ref):
    f32 = jnp.float32
    logits = logits_ref[...]
    mask = (masks_ref[...] == 1).astype(f32)
    p = jax.nn.softmax(logits, axis=1)
    g0 = p * mask
    inactive = p * (1.0 - mask)
    inorm = inactive / (jnp.sum(inactive, axis=1, keepdims=True) + EPS)
    se = jnp.zeros((B, LOW), f32)
    for e in range(E):
        se = se + selemb_ref[:, e, :] * inorm[:, e:e + 1]
    se_ref[...] = se

    iota_e = jax.lax.broadcasted_iota(jnp.int32, (B, E), 1)
    m1 = jnp.max(g0, axis=1, keepdims=True)
    i1 = jnp.min(jnp.where(g0 == m1, iota_e, E), axis=1, keepdims=True)
    g1 = jnp.where(iota_e == i1, -1.0, g0)
    m2 = jnp.max(g1, axis=1, keepdims=True)
    i2 = jnp.min(jnp.where(g1 == m2, iota_e, E), axis=1, keepdims=True)
    de = m1 + m2 + EPS
    w1 = m1 / de
    w2 = m2 / de
    gcomb = jnp.where(iota_e == i1, w1, 0.0) + jnp.where(iota_e == i2, w2, 0.0)
    gb_ref[...] = jnp.dot(gcomb, bias_ref[...], preferred_element_type=f32)

    logits_t = jnp.transpose(logits, (1, 0))
    mask_t = jnp.transpose(mask, (1, 0))
    p_t = jax.nn.softmax(logits_t, axis=0)
    g0t = p_t * mask_t
    iota_et = jax.lax.broadcasted_iota(jnp.int32, (E, B), 0)
    m1t = jnp.max(g0t, axis=0, keepdims=True)
    i1t = jnp.min(jnp.where(g0t == m1t, iota_et, E), axis=0, keepdims=True)
    g1t = jnp.where(iota_et == i1t, -1.0, g0t)
    m2t = jnp.max(g1t, axis=0, keepdims=True)
    i2t = jnp.min(jnp.where(g1t == m2t, iota_et, E), axis=0, keepdims=True)
    det = m1t + m2t + EPS
    w1t = m1t / det
    w2t = m2t / det

    e_row = jnp.concatenate([i1t, i2t], axis=1)
    w_row = jnp.concatenate([w1t, w2t], axis=1)
    b_row = jax.lax.broadcasted_iota(jnp.int32, (1, NPAIR), 1) % B
    gw_ref[...] = w_row

    onehot = (jnp.broadcast_to(e_row, (E, NPAIR)) ==
              jax.lax.broadcasted_iota(jnp.int32, (E, NPAIR), 0)).astype(f32)
    qi = jax.lax.broadcasted_iota(jnp.int32, (NPAIR, NPAIR), 0)
    qj = jax.lax.broadcasted_iota(jnp.int32, (NPAIR, NPAIR), 1)
    upper = (qi < qj).astype(f32)
    prefix = jnp.dot(onehot, upper, preferred_element_type=f32)
    rank = jnp.sum(onehot * prefix, axis=0, keepdims=True)

    cnt_col = jnp.sum(onehot, axis=1, keepdims=True)
    cntpad_col = jnp.floor((cnt_col + (P - 1)) * (1.0 / P)) * P
    eye = (jax.lax.broadcasted_iota(jnp.int32, (E, E), 0) ==
           jax.lax.broadcasted_iota(jnp.int32, (E, E), 1)).astype(f32)
    cntpad_row = jnp.dot(jnp.ones((1, E), f32), cntpad_col * eye,
                         preferred_element_type=f32)
    u8 = (jax.lax.broadcasted_iota(jnp.int32, (E, E), 0) <
          jax.lax.broadcasted_iota(jnp.int32, (E, E), 1)).astype(f32)
    offpad_row = jnp.dot(cntpad_row, u8, preferred_element_type=f32)
    endpad_row = offpad_row + cntpad_row

    l8 = (jax.lax.broadcasted_iota(jnp.int32, (E, E), 0) >
          jax.lax.broadcasted_iota(jnp.int32, (E, E), 1)).astype(f32)
    offpad_col = jnp.dot(l8, cntpad_col, preferred_element_type=f32)
    off_sel = jnp.sum(onehot * jnp.broadcast_to(offpad_col, (E, NPAIR)),
                      axis=0, keepdims=True)
    pos = off_sel + rank
    pos_ref[...] = pos.astype(jnp.int32)

    slot = jax.lax.broadcasted_iota(jnp.int32, (NS, NPAIR), 0).astype(f32)
    hit = (slot == jnp.broadcast_to(pos, (NS, NPAIR))).astype(f32)
    sid_ref[...] = jnp.sum(hit * b_row.astype(f32), axis=1,
                           keepdims=True).astype(jnp.int32)

    blk = (jax.lax.broadcasted_iota(jnp.int32, (NB, E), 0) * P).astype(f32)
    endb = jnp.broadcast_to(endpad_row, (NB, E))
    ebid = jnp.sum((blk >= endb).astype(f32), axis=1, keepdims=True)
    ebid_ref[...] = jnp.minimum(ebid, E - 1).astype(jnp.int32)
    nblk_ref[...] = (jnp.sum(cntpad_col * (1.0 / P))
                     .reshape(1, 1).astype(jnp.int32))


def _moe_kernel(sid_ref, ebid_ref, nblk_ref, pos_ref, gw_ref,
                x_ref, wt_ref, gb_ref, out_ref,
                xga_ref, xgb_ref, ys_ref):
    f32 = jnp.float32
    nb = nblk_ref[0, 0]

    def gather_block(j, buf_ref):
        for p_i in range(P):
            s = sid_ref[j * P + p_i, 0]
            buf_ref[pl.ds(p_i * R, R), :] = x_ref[s]

    def dot_block(j, buf_ref):
        e = ebid_ref[j, 0]
        y = jax.lax.dot_general(
            buf_ref[...], wt_ref[e], (((1,), (0,)), ((), ())),
            preferred_element_type=f32)
        for p_i in range(P):
            ys_ref[pl.ds((j * P + p_i) * RPAD, R), :] = (
                y[p_i * R:(p_i + 1) * R, :].astype(jnp.bfloat16))

    gather_block(0, xga_ref)

    def pair_body(t, carry):
        j0 = 2 * t
        j1 = j0 + 1

        @pl.when(j1 < nb)
        def _prefetch_b():
            gather_block(j1, xgb_ref)

        dot_block(j0, xga_ref)

        @pl.when(j1 < nb)
        def _second():
            @pl.when(j1 + 1 < nb)
            def _prefetch_a():
                gather_block(j1 + 1, xga_ref)

            dot_block(j1, xgb_ref)

        return carry

    jax.lax.fori_loop(0, (nb + 1) // 2, pair_body, 0)

    def combine_body(bi, carry):
        p0 = pos_ref[0, bi]
        p1 = pos_ref[0, B + bi]
        g0 = gw_ref[0, bi]
        g1 = gw_ref[0, B + bi]
        y0 = ys_ref[pl.ds(p0 * RPAD, R), :].astype(f32)
        y1 = ys_ref[pl.ds(p1 * RPAD, R), :].astype(f32)
        out_ref[bi] = (gb_ref[bi][None, :] + g0 * y0
                       + g1 * y1).astype(jnp.bfloat16)
        return carry

    jax.lax.fori_loop(0, B, combine_body, 0)


@functools.partial(jax.jit, static_argnames=())
def kernel(cycle_curve_data, logits, moe_masks, selection_embeddings, W, b):
    f32 = jnp.float32
    x3 = cycle_curve_data.reshape(B, R, IN_F).astype(jnp.bfloat16)
    wt = jnp.transpose(W, (0, 2, 1)).astype(jnp.bfloat16)

    se, gb, sid, ebid, nblk, pos, gw = pl.pallas_call(
        _routing_kernel,
        out_shape=(
            jax.ShapeDtypeStruct((B, LOW), f32),
            jax.ShapeDtypeStruct((B, D_MODEL), f32),
            jax.ShapeDtypeStruct((NS, 1), jnp.int32),
            jax.ShapeDtypeStruct((NB, 1), jnp.int32),
            jax.ShapeDtypeStruct((1, 1), jnp.int32),
            jax.ShapeDtypeStruct((1, NPAIR), jnp.int32),
            jax.ShapeDtypeStruct((1, NPAIR), f32),
        ),
    )(logits, moe_masks, selection_embeddings, b)

    out = pl.pallas_call(
        _moe_kernel,
        in_specs=[
            pl.BlockSpec(memory_space=pltpu.SMEM),
            pl.BlockSpec(memory_space=pltpu.SMEM),
            pl.BlockSpec(memory_space=pltpu.SMEM),
            pl.BlockSpec(memory_space=pltpu.SMEM),
            pl.BlockSpec(memory_space=pltpu.SMEM),
            pl.BlockSpec(memory_space=pltpu.VMEM),
            pl.BlockSpec(memory_space=pltpu.VMEM),
            pl.BlockSpec(memory_space=pltpu.VMEM),
        ],
        out_shape=jax.ShapeDtypeStruct((B, R, D_MODEL), jnp.bfloat16),
        scratch_shapes=[
            pltpu.VMEM((PR, IN_F), jnp.bfloat16),
            pltpu.VMEM((PR, IN_F), jnp.bfloat16),
            pltpu.VMEM((NS * RPAD, D_MODEL), jnp.bfloat16),
        ],
    )(sid, ebid, nblk, pos, gw, x3, wt, gb)

    final_out = out.reshape(B, L, C, D_MODEL)
    return (final_out, jnp.float32(0.0), se)


# speedup vs baseline: 1.2918x; 1.0592x over previous
import functools

import jax
import jax.numpy as jnp
from jax.experimental import pallas as pl
from jax.experimental.pallas import tpu as pltpu

B = 128
L = 8
C = 3
R = L * C
IN_F = 900
D_MODEL = 768
E = 8
TOP_K = 2
LOW = 64
EPS = 1e-09

P = 16
PR = P * R
RPAD = 32
NPAIR = B * TOP_K
NS = ((NPAIR + E * (P - 1) + P - 1) // P) * P
NB = NS // P


def _routing_kernel(logits_ref, masks_ref, selemb_ref, bias_ref,
                    se_ref, gb_ref, sid_ref, ebid_ref, nblk_ref,
                    pos_ref, gw_ref):
    f32 = jnp.float32
    logits = logits_ref[...]
    mask = (masks_ref[...] == 1).astype(f32)
    p = jax.nn.softmax(logits, axis=1)
    g0 = p * mask
    inactive = p * (1.0 - mask)
    inorm = inactive / (jnp.sum(inactive, axis=1, keepdims=True) + EPS)
    se = jnp.zeros((B, LOW), f32)
    for e in range(E):
        se = se + selemb_ref[:, e, :] * inorm[:, e:e + 1]
    se_ref[...] = se

    iota_e = jax.lax.broadcasted_iota(jnp.int32, (B, E), 1)
    m1 = jnp.max(g0, axis=1, keepdims=True)
    i1 = jnp.min(jnp.where(g0 == m1, iota_e, E), axis=1, keepdims=True)
    g1 = jnp.where(iota_e == i1, -1.0, g0)
    m2 = jnp.max(g1, axis=1, keepdims=True)
    i2 = jnp.min(jnp.where(g1 == m2, iota_e, E), axis=1, keepdims=True)
    de = m1 + m2 + EPS
    w1 = m1 / de
    w2 = m2 / de
    gcomb = jnp.where(iota_e == i1, w1, 0.0) + jnp.where(iota_e == i2, w2, 0.0)
    gb_ref[...] = jnp.dot(gcomb, bias_ref[...], preferred_element_type=f32)

    logits_t = jnp.transpose(logits, (1, 0))
    mask_t = jnp.transpose(mask, (1, 0))
    p_t = jax.nn.softmax(logits_t, axis=0)
    g0t = p_t * mask_t
    iota_et = jax.lax.broadcasted_iota(jnp.int32, (E, B), 0)
    m1t = jnp.max(g0t, axis=0, keepdims=True)
    i1t = jnp.min(jnp.where(g0t == m1t, iota_et, E), axis=0, keepdims=True)
    g1t = jnp.where(iota_et == i1t, -1.0, g0t)
    m2t = jnp.max(g1t, axis=0, keepdims=True)
    i2t = jnp.min(jnp.where(g1t == m2t, iota_et, E), axis=0, keepdims=True)
    det = m1t + m2t + EPS
    w1t = m1t / det
    w2t = m2t / det

    e_row = jnp.concatenate([i1t, i2t], axis=1)
    w_row = jnp.concatenate([w1t, w2t], axis=1)
    b_row = jax.lax.broadcasted_iota(jnp.int32, (1, NPAIR), 1) % B
    gw_ref[...] = w_row

    onehot = (jnp.broadcast_to(e_row, (E, NPAIR)) ==
              jax.lax.broadcasted_iota(jnp.int32, (E, NPAIR), 0)).astype(f32)
    qi = jax.lax.broadcasted_iota(jnp.int32, (NPAIR, NPAIR), 0)
    qj = jax.lax.broadcasted_iota(jnp.int32, (NPAIR, NPAIR), 1)
    upper = (qi < qj).astype(f32)
    prefix = jnp.dot(onehot, upper, preferred_element_type=f32)
    rank = jnp.sum(onehot * prefix, axis=0, keepdims=True)

    cnt_col = jnp.sum(onehot, axis=1, keepdims=True)
    cntpad_col = jnp.floor((cnt_col + (P - 1)) * (1.0 / P)) * P
    eye = (jax.lax.broadcasted_iota(jnp.int32, (E, E), 0) ==
           jax.lax.broadcasted_iota(jnp.int32, (E, E), 1)).astype(f32)
    cntpad_row = jnp.dot(jnp.ones((1, E), f32), cntpad_col * eye,
                         preferred_element_type=f32)
    u8 = (jax.lax.broadcasted_iota(jnp.int32, (E, E), 0) <
          jax.lax.broadcasted_iota(jnp.int32, (E, E), 1)).astype(f32)
    offpad_row = jnp.dot(cntpad_row, u8, preferred_element_type=f32)
    endpad_row = offpad_row + cntpad_row

    l8 = (jax.lax.broadcasted_iota(jnp.int32, (E, E), 0) >
          jax.lax.broadcasted_iota(jnp.int32, (E, E), 1)).astype(f32)
    offpad_col = jnp.dot(l8, cntpad_col, preferred_element_type=f32)
    off_sel = jnp.sum(onehot * jnp.broadcast_to(offpad_col, (E, NPAIR)),
                      axis=0, keepdims=True)
    pos = off_sel + rank
    pos_ref[...] = pos.astype(jnp.int32)

    slot = jax.lax.broadcasted_iota(jnp.int32, (NS, NPAIR), 0).astype(f32)
    hit = (slot == jnp.broadcast_to(pos, (NS, NPAIR))).astype(f32)
    sid_ref[...] = jnp.sum(hit * b_row.astype(f32), axis=1,
                           keepdims=True).astype(jnp.int32)

    blk = (jax.lax.broadcasted_iota(jnp.int32, (NB, E), 0) * P).astype(f32)
    endb = jnp.broadcast_to(endpad_row, (NB, E))
    ebid = jnp.sum((blk >= endb).astype(f32), axis=1, keepdims=True)
    ebid_ref[...] = jnp.minimum(ebid, E - 1).astype(jnp.int32)
    nblk_ref[...] = (jnp.sum(cntpad_col * (1.0 / P))
                     .reshape(1, 1).astype(jnp.int32))


def _moe_kernel(sid_ref, ebid_ref, nblk_ref, pos_ref, gw_ref,
                x_ref, wt_ref, gb_ref, out_ref,
                xga_ref, xgb_ref, ys_ref):
    f32 = jnp.float32
    nb = nblk_ref[0, 0]

    def gather_block(j, buf_ref):
        for p_i in range(P):
            s = sid_ref[j * P + p_i, 0]
            buf_ref[pl.ds(p_i * R, R), :] = x_ref[s]

    def dot_block(j, buf_ref):
        e = ebid_ref[j, 0]
        y = jax.lax.dot_general(
            buf_ref[...], wt_ref[e], (((1,), (0,)), ((), ())),
            preferred_element_type=f32)
        for p_i in range(P):
            ys_ref[pl.ds((j * P + p_i) * RPAD, R), :] = (
                y[p_i * R:(p_i + 1) * R, :].astype(jnp.bfloat16))

    gather_block(0, xga_ref)

    def pair_body(t, carry):
        j0 = 2 * t
        j1 = j0 + 1

        @pl.when(j1 < nb)
        def _prefetch_b():
            gather_block(j1, xgb_ref)

        dot_block(j0, xga_ref)

        @pl.when(j1 < nb)
        def _second():
            @pl.when(j1 + 1 < nb)
            def _prefetch_a():
                gather_block(j1 + 1, xga_ref)

            dot_block(j1, xgb_ref)

        return carry

    jax.lax.fori_loop(0, (nb + 1) // 2, pair_body, 0)

    def combine_body(bi, carry):
        p0 = pos_ref[0, bi]
        p1 = pos_ref[0, B + bi]
        g0 = gw_ref[0, bi]
        g1 = gw_ref[0, B + bi]
        y0 = ys_ref[pl.ds(p0 * RPAD, R), :].astype(f32)
        y1 = ys_ref[pl.ds(p1 * RPAD, R), :].astype(f32)
        out_ref[bi] = (gb_ref[bi][None, :] + g0 * y0
                       + g1 * y1).astype(jnp.bfloat16)
        return carry

    jax.lax.fori_loop(0, B, combine_body, 0)


@functools.partial(jax.jit, static_argnames=())
def kernel(cycle_curve_data, logits, moe_masks, selection_embeddings, W, b):
    f32 = jnp.float32
    x3 = cycle_curve_data.reshape(B, R, IN_F).astype(jnp.bfloat16)
    wt = jnp.transpose(W, (0, 2, 1)).astype(jnp.bfloat16)

    se, gb, sid, ebid, nblk, pos, gw = pl.pallas_call(
        _routing_kernel,
        out_shape=(
            jax.ShapeDtypeStruct((B, LOW), f32),
            jax.ShapeDtypeStruct((B, D_MODEL), f32),
            jax.ShapeDtypeStruct((NS, 1), jnp.int32),
            jax.ShapeDtypeStruct((NB, 1), jnp.int32),
            jax.ShapeDtypeStruct((1, 1), jnp.int32),
            jax.ShapeDtypeStruct((1, NPAIR), jnp.int32),
            jax.ShapeDtypeStruct((1, NPAIR), f32),
        ),
    )(logits, moe_masks, selection_embeddings, b)

    out = pl.pallas_call(
        _moe_kernel,
        in_specs=[
            pl.BlockSpec(memory_space=pltpu.SMEM),
            pl.BlockSpec(memory_space=pltpu.SMEM),
            pl.BlockSpec(memory_space=pltpu.SMEM),
            pl.BlockSpec(memory_space=pltpu.SMEM),
            pl.BlockSpec(memory_space=pltpu.SMEM),
            pl.BlockSpec(memory_space=pltpu.VMEM),
            pl.BlockSpec(memory_space=pltpu.VMEM),
            pl.BlockSpec(memory_space=pltpu.VMEM),
        ],
        out_shape=jax.ShapeDtypeStruct((B, R, D_MODEL), jnp.bfloat16),
        scratch_shapes=[
            pltpu.VMEM((PR, IN_F), jnp.bfloat16),
            pltpu.VMEM((PR, IN_F), jnp.bfloat16),
            pltpu.VMEM((NS * RPAD, D_MODEL), jnp.bfloat16),
        ],
    )(sid, ebid, nblk, pos, gw, x3, wt, gb)

    final_out = out.reshape(B, L, C, D_MODEL)
    return (final_out, jnp.float32(0.0), se)


# speedup vs baseline: 1.2992x; 1.0058x over previous
import functools

import jax
import jax.numpy as jnp
from jax.experimental import pallas as pl
from jax.experimental.pallas import tpu as pltpu

B = 128
L = 8
C = 3
R = L * C
IN_F = 900
D_MODEL = 768
E = 8
TOP_K = 2
LOW = 64
EPS = 1e-09

P = 32
PR = P * R
RPAD = 32
NPAIR = B * TOP_K
NS = ((NPAIR + E * (P - 1) + P - 1) // P) * P
NB = NS // P


def _routing_kernel(logits_ref, masks_ref, selemb_ref, bias_ref,
                    se_ref, gb_ref, sid_ref, ebid_ref, nblk_ref,
                    pos_ref, gw_ref):
    f32 = jnp.float32
    logits = logits_ref[...]
    mask = (masks_ref[...] == 1).astype(f32)
    p = jax.nn.softmax(logits, axis=1)
    g0 = p * mask
    inactive = p * (1.0 - mask)
    inorm = inactive / (jnp.sum(inactive, axis=1, keepdims=True) + EPS)
    se = jnp.zeros((B, LOW), f32)
    for e in range(E):
        se = se + selemb_ref[:, e, :] * inorm[:, e:e + 1]
    se_ref[...] = se

    iota_e = jax.lax.broadcasted_iota(jnp.int32, (B, E), 1)
    m1 = jnp.max(g0, axis=1, keepdims=True)
    i1 = jnp.min(jnp.where(g0 == m1, iota_e, E), axis=1, keepdims=True)
    g1 = jnp.where(iota_e == i1, -1.0, g0)
    m2 = jnp.max(g1, axis=1, keepdims=True)
    i2 = jnp.min(jnp.where(g1 == m2, iota_e, E), axis=1, keepdims=True)
    de = m1 + m2 + EPS
    w1 = m1 / de
    w2 = m2 / de
    gcomb = jnp.where(iota_e == i1, w1, 0.0) + jnp.where(iota_e == i2, w2, 0.0)
    gb_ref[...] = jnp.dot(gcomb, bias_ref[...], preferred_element_type=f32)

    logits_t = jnp.transpose(logits, (1, 0))
    mask_t = jnp.transpose(mask, (1, 0))
    p_t = jax.nn.softmax(logits_t, axis=0)
    g0t = p_t * mask_t
    iota_et = jax.lax.broadcasted_iota(jnp.int32, (E, B), 0)
    m1t = jnp.max(g0t, axis=0, keepdims=True)
    i1t = jnp.min(jnp.where(g0t == m1t, iota_et, E), axis=0, keepdims=True)
    g1t = jnp.where(iota_et == i1t, -1.0, g0t)
    m2t = jnp.max(g1t, axis=0, keepdims=True)
    i2t = jnp.min(jnp.where(g1t == m2t, iota_et, E), axis=0, keepdims=True)
    det = m1t + m2t + EPS
    w1t = m1t / det
    w2t = m2t / det

    e_row = jnp.concatenate([i1t, i2t], axis=1)
    w_row = jnp.concatenate([w1t, w2t], axis=1)
    b_row = jax.lax.broadcasted_iota(jnp.int32, (1, NPAIR), 1) % B
    gw_ref[...] = w_row

    onehot = (jnp.broadcast_to(e_row, (E, NPAIR)) ==
              jax.lax.broadcasted_iota(jnp.int32, (E, NPAIR), 0)).astype(f32)
    qi = jax.lax.broadcasted_iota(jnp.int32, (NPAIR, NPAIR), 0)
    qj = jax.lax.broadcasted_iota(jnp.int32, (NPAIR, NPAIR), 1)
    upper = (qi < qj).astype(f32)
    prefix = jnp.dot(onehot, upper, preferred_element_type=f32)
    rank = jnp.sum(onehot * prefix, axis=0, keepdims=True)

    cnt_col = jnp.sum(onehot, axis=1, keepdims=True)
    cntpad_col = jnp.floor((cnt_col + (P - 1)) * (1.0 / P)) * P
    eye = (jax.lax.broadcasted_iota(jnp.int32, (E, E), 0) ==
           jax.lax.broadcasted_iota(jnp.int32, (E, E), 1)).astype(f32)
    cntpad_row = jnp.dot(jnp.ones((1, E), f32), cntpad_col * eye,
                         preferred_element_type=f32)
    u8 = (jax.lax.broadcasted_iota(jnp.int32, (E, E), 0) <
          jax.lax.broadcasted_iota(jnp.int32, (E, E), 1)).astype(f32)
    offpad_row = jnp.dot(cntpad_row, u8, preferred_element_type=f32)
    endpad_row = offpad_row + cntpad_row

    l8 = (jax.lax.broadcasted_iota(jnp.int32, (E, E), 0) >
          jax.lax.broadcasted_iota(jnp.int32, (E, E), 1)).astype(f32)
    offpad_col = jnp.dot(l8, cntpad_col, preferred_element_type=f32)
    off_sel = jnp.sum(onehot * jnp.broadcast_to(offpad_col, (E, NPAIR)),
                      axis=0, keepdims=True)
    pos = off_sel + rank
    pos_ref[...] = pos.astype(jnp.int32)

    slot = jax.lax.broadcasted_iota(jnp.int32, (NS, NPAIR), 0).astype(f32)
    hit = (slot == jnp.broadcast_to(pos, (NS, NPAIR))).astype(f32)
    sid_ref[...] = jnp.sum(hit * b_row.astype(f32), axis=1,
                           keepdims=True).astype(jnp.int32)

    blk = (jax.lax.broadcasted_iota(jnp.int32, (NB, E), 0) * P).astype(f32)
    endb = jnp.broadcast_to(endpad_row, (NB, E))
    ebid = jnp.sum((blk >= endb).astype(f32), axis=1, keepdims=True)
    ebid_ref[...] = jnp.minimum(ebid, E - 1).astype(jnp.int32)
    nblk_ref[...] = (jnp.sum(cntpad_col * (1.0 / P))
                     .reshape(1, 1).astype(jnp.int32))


def _moe_kernel(sid_ref, ebid_ref, nblk_ref, pos_ref, gw_ref,
                x_ref, wt_ref, gb_ref, out_ref,
                xga_ref, xgb_ref, ys_ref):
    f32 = jnp.float32
    nb = nblk_ref[0, 0]

    def gather_block(j, buf_ref):
        for p_i in range(P):
            s = sid_ref[j * P + p_i, 0]
            buf_ref[pl.ds(p_i * R, R), :] = x_ref[s]

    def dot_block(j, buf_ref):
        e = ebid_ref[j, 0]
        y = jax.lax.dot_general(
            buf_ref[...], wt_ref[e], (((1,), (0,)), ((), ())),
            preferred_element_type=f32)
        for p_i in range(P):
            ys_ref[pl.ds((j * P + p_i) * RPAD, R), :] = (
                y[p_i * R:(p_i + 1) * R, :].astype(jnp.bfloat16))

    gather_block(0, xga_ref)

    def pair_body(t, carry):
        j0 = 2 * t
        j1 = j0 + 1

        @pl.when(j1 < nb)
        def _prefetch_b():
            gather_block(j1, xgb_ref)

        dot_block(j0, xga_ref)

        @pl.when(j1 < nb)
        def _second():
            @pl.when(j1 + 1 < nb)
            def _prefetch_a():
                gather_block(j1 + 1, xga_ref)

            dot_block(j1, xgb_ref)

        return carry

    jax.lax.fori_loop(0, (nb + 1) // 2, pair_body, 0)

    def combine_body(bi, carry):
        p0 = pos_ref[0, bi]
        p1 = pos_ref[0, B + bi]
        g0 = gw_ref[0, bi]
        g1 = gw_ref[0, B + bi]
        y0 = ys_ref[pl.ds(p0 * RPAD, R), :].astype(f32)
        y1 = ys_ref[pl.ds(p1 * RPAD, R), :].astype(f32)
        out_ref[bi] = (gb_ref[bi][None, :] + g0 * y0
                       + g1 * y1).astype(jnp.bfloat16)
        return carry

    jax.lax.fori_loop(0, B, combine_body, 0)


@functools.partial(jax.jit, static_argnames=())
def kernel(cycle_curve_data, logits, moe_masks, selection_embeddings, W, b):
    f32 = jnp.float32
    x3 = cycle_curve_data.reshape(B, R, IN_F).astype(jnp.bfloat16)
    wt = jnp.transpose(W, (0, 2, 1)).astype(jnp.bfloat16)

    se, gb, sid, ebid, nblk, pos, gw = pl.pallas_call(
        _routing_kernel,
        out_shape=(
            jax.ShapeDtypeStruct((B, LOW), f32),
            jax.ShapeDtypeStruct((B, D_MODEL), f32),
            jax.ShapeDtypeStruct((NS, 1), jnp.int32),
            jax.ShapeDtypeStruct((NB, 1), jnp.int32),
            jax.ShapeDtypeStruct((1, 1), jnp.int32),
            jax.ShapeDtypeStruct((1, NPAIR), jnp.int32),
            jax.ShapeDtypeStruct((1, NPAIR), f32),
        ),
    )(logits, moe_masks, selection_embeddings, b)

    out = pl.pallas_call(
        _moe_kernel,
        in_specs=[
            pl.BlockSpec(memory_space=pltpu.SMEM),
            pl.BlockSpec(memory_space=pltpu.SMEM),
            pl.BlockSpec(memory_space=pltpu.SMEM),
            pl.BlockSpec(memory_space=pltpu.SMEM),
            pl.BlockSpec(memory_space=pltpu.SMEM),
            pl.BlockSpec(memory_space=pltpu.VMEM),
            pl.BlockSpec(memory_space=pltpu.VMEM),
            pl.BlockSpec(memory_space=pltpu.VMEM),
        ],
        out_shape=jax.ShapeDtypeStruct((B, R, D_MODEL), jnp.bfloat16),
        scratch_shapes=[
            pltpu.VMEM((PR, IN_F), jnp.bfloat16),
            pltpu.VMEM((PR, IN_F), jnp.bfloat16),
            pltpu.VMEM((NS * RPAD, D_MODEL), jnp.bfloat16),
        ],
    )(sid, ebid, nblk, pos, gw, x3, wt, gb)

    final_out = out.reshape(B, L, C, D_MODEL)
    return (final_out, jnp.float32(0.0), se)
